# baseline (speedup 1.0000x reference)
import functools

import jax
import jax.numpy as jnp
from jax import lax
from jax.experimental import pallas as pl
from jax.experimental.pallas import tpu as pltpu
from jax.experimental.pallas import tpu_sc as plsc

L = 5
H = 128
TXT = 768
OUT = 512
NN = 10000
NE = 320000
B = 128
NBOND = 5
NATOM = 118

NC = 2
NS = 16
NW = NC * NS

ECHUNK = 64
NE_PAD = 327680
NCHT = NE_PAD // ECHUNK
ECH0 = 320
ECH1 = 0
NBUF = 5
GDEPTH = 3

NACC = 10112
ROWS_PER_SUB = NACC // NS

NPT = 320

TILE = 1000
NTILES = NN // TILE

_f32 = jnp.float32


_INV_SQRT2 = 0.7071067811865476


def _gelu(x):
    return 0.5 * x * (1.0 + lax.erf(x * _INV_SQRT2))


def _ln(x, g=None, b=None, eps=1e-5):
    mu = jnp.mean(x, axis=-1, keepdims=True)
    var = jnp.var(x, axis=-1, keepdims=True)
    y = (x - mu) / jnp.sqrt(var + eps)
    if g is not None:
        y = y * g + b
    return y


EGRP = 40


def _sc_body(do_pool, pk_hbm, gtab_hbm, hin_hbm, batch_hbm,
             agg_hbm, pool_hbm,
             pk_v, gidx_b, didx_b, buf_v, bat_v, acc_sh,
             gs0, gs1, gs2, gs3, gs4, ss0, ss1, ss2, ss3, ss4):
    c = lax.axis_index("c")
    s = lax.axis_index("s")
    wid = c * NS + s
    gsems = [gs0, gs1, gs2, gs3, gs4]
    ssems = [ss0, ss1, ss2, ss3, ss4]
    rowb = [buf_v.at[pl.ds(u * ECHUNK, ECHUNK)] for u in range(NBUF)]
    part = buf_v.at[pl.ds(0, B)]
    hrow = buf_v.at[pl.ds(128, 64)]

    def _zrow(i, _):
        for k in range(8):
            buf_v[i, pl.ds(k * 16, 16)] = jnp.zeros((16,), _f32)
        return 0
    lax.fori_loop(0, NPT, _zrow, 0)
    pltpu.sync_copy(buf_v.at[pl.ds(0, NPT)],
                    acc_sh.at[pl.ds(s * ROWS_PER_SUB, NPT)])
    pltpu.sync_copy(buf_v.at[pl.ds(0, ROWS_PER_SUB - NPT)],
                    acc_sh.at[pl.ds(s * ROWS_PER_SUB + NPT,
                                    ROWS_PER_SUB - NPT)])
    plsc.subcore_barrier()

    if do_pool:
        base = jnp.minimum(wid * NPT, NN - NPT)
        pltpu.sync_copy(batch_hbm.at[pl.ds(base, NPT)], bat_v.at[pl.ds(0, NPT)])

        def _pinit(i, _):
            for k in range(8):
                part[i, pl.ds(k * 16, 16)] = jnp.full((16,), -1e30, _f32)
            return 0
        lax.fori_loop(0, B, _pinit, 0)

        def _pchunk(cc, _):
            pltpu.sync_copy(hin_hbm.at[pl.ds(base + cc * 64, 64)], hrow)

            def _prow(i, _):
                g = bat_v[pl.ds(cc * 64 + i, 16)][0]
                for k in range(8):
                    sl = pl.ds(k * 16, 16)
                    part[g, sl] = jnp.maximum(part[g, sl], hrow[i, sl])
                return 0
            lax.fori_loop(0, 64, _prow, 0)
            return 0
        lax.fori_loop(0, NPT // 64, _pchunk, 0)
        pltpu.sync_copy(part, pool_hbm.at[wid])

    cbase = jnp.where(c == 0, s * ECH0, NS * ECH0 + s * ECH1)
    ngrp = jnp.where(c == 0, ECH0 // EGRP, ECH1 // EGRP)

    def _unpack(lj, u):
        for k in range(ECHUNK // 16):
            sl = pl.ds(k * 16, 16)
            v = pk_v[lj, sl]
            gidx_b[u, sl] = v & 0x1FFFF
            didx_b[u, sl] = v >> 17

    def _egrp(grp, _):
        off = cbase + grp * EGRP
        pltpu.sync_copy(pk_hbm.at[pl.ds(off, EGRP)], pk_v)
        for u in range(GDEPTH):
            _unpack(u, u)
            pltpu.async_copy(gtab_hbm.at[gidx_b.at[u]], rowb[u], gsems[u])

        def _emac(m, _):
            for u in range(NBUF):
                lj = m * NBUF + u
                pltpu.make_async_copy(gtab_hbm.at[gidx_b.at[u]], rowb[u],
                                      gsems[u]).wait()
                pltpu.async_copy(rowb[u], acc_sh.at[didx_b.at[u]],
                                 ssems[u], add=True)
                bt = (u + GDEPTH) % NBUF
                lt = lj + GDEPTH

                @pl.when(lt < EGRP)
                def _():
                    @pl.when(lt >= NBUF)
                    def _():
                        pltpu.make_async_copy(
                            rowb[bt], acc_sh.at[didx_b.at[bt]],
                            ssems[bt]).wait()
                    _unpack(lt, bt)
                    pltpu.async_copy(gtab_hbm.at[gidx_b.at[bt]], rowb[bt],
                                     gsems[bt])
            return 0

        lax.fori_loop(0, EGRP // NBUF, _emac, 0)
        for u in range(NBUF):
            pltpu.make_async_copy(rowb[u], acc_sh.at[didx_b.at[u]],
                                  ssems[u]).wait()
        return 0

    lax.fori_loop(0, ngrp, _egrp, 0)
    plsc.subcore_barrier()

    pltpu.sync_copy(acc_sh.at[pl.ds(s * ROWS_PER_SUB, ROWS_PER_SUB)],
                    agg_hbm.at[c, pl.ds(s * ROWS_PER_SUB, ROWS_PER_SUB)])


def _sc_edge_stage(packed2d, gtab, hin, batch, do_pool):
    mesh = plsc.VectorSubcoreMesh(core_axis_name="c", subcore_axis_name="s")
    fn = pl.kernel(
        functools.partial(_sc_body, do_pool),
        out_type=[
            jax.ShapeDtypeStruct((NC, NACC, H), _f32),
            jax.ShapeDtypeStruct((NW, B, H), _f32),
        ],
        mesh=mesh,
        scratch_types=(
            [
                pltpu.VMEM((EGRP, ECHUNK), jnp.int32),
                pltpu.VMEM((NBUF, ECHUNK), jnp.int32),
                pltpu.VMEM((NBUF, ECHUNK), jnp.int32),
                pltpu.VMEM((NBUF * ECHUNK, H), _f32),
                pltpu.VMEM((NPT + 16,), jnp.int32),
                pltpu.VMEM_SHARED((NACC, H), _f32),
            ]
            + [pltpu.SemaphoreType.DMA] * (2 * NBUF)
        ),
    )
    return fn(packed2d, gtab, hin, batch)


def _tc0_body(aoh_ref, aemb_ref, vnrow_ref, bond_ref,
              hin_ref, g_ref, vn0_ref):
    pi = pl.program_id(0)
    h = jnp.dot(aoh_ref[...], aemb_ref[...],
                preferred_element_type=_f32) + vnrow_ref[...]
    hin_ref[...] = h
    for bb in range(NBOND):
        g_ref[bb] = _gelu(h + bond_ref[bb])

    @pl.when(pi == 0)
    def _():
        vn0_ref[...] = jnp.broadcast_to(vnrow_ref[...], (B, H))


def _tc0_stage(atom_oh, atom_embp, vn_row, bond0):
    return pl.pallas_call(
        _tc0_body,
        grid=(NTILES,),
        in_specs=[
            pl.BlockSpec((TILE, 128), lambda i: (i, 0)),
            pl.BlockSpec((128, H), lambda i: (0, 0)),
            pl.BlockSpec((1, H), lambda i: (0, 0)),
            pl.BlockSpec((NBOND, 1, H), lambda i: (0, 0, 0)),
        ],
        out_specs=[
            pl.BlockSpec((TILE, H), lambda i: (i, 0)),
            pl.BlockSpec((NBOND, TILE, H), lambda i: (0, i, 0)),
            pl.BlockSpec((B, H), lambda i: (0, 0)),
        ],
        out_shape=[
            jax.ShapeDtypeStruct((NN, H), _f32),
            jax.ShapeDtypeStruct((NBOND, NN, H), _f32),
            jax.ShapeDtypeStruct((B, H), _f32),
        ],
    )(atom_oh, atom_embp, vn_row, bond0)


def _dense_hn(last, hin, agg, oh, ssg, eps, w1, b1, lng, lnb, w2, b2):
    z = (1.0 + eps) * hin + agg
    z = jnp.dot(z, w1, preferred_element_type=_f32) + b1
    z = _ln(z, lng, lnb)
    z = jnp.dot(_gelu(z), w2, preferred_element_type=_f32) + b2
    hn = _ln(z)
    mod = jnp.dot(oh, ssg, preferred_element_type=_f32)
    shift = mod[:, :H]
    scale = mod[:, H:2 * H]
    gate = mod[:, 2 * H:]
    hn = hn * (1.0 + scale) + shift
    if not last:
        hn = _gelu(hn)
    return gate * hn + hin


def _tcmid_body(hin_ref, agg_ref, pool_ref, oh_ref, c_ref, aw_ref, ab_ref,
                eps_ref, w1_ref, b1_ref, lng_ref, lnb_ref, w2_ref, b2_ref,
                vw1_ref, vb1_ref, vlng_ref, vlnb_ref, vw2_ref, vb2_ref,
                vn_ref, bondn_ref,
                hinn_ref, gn_ref, vnn_ref,
                vn_scr, ssg_scr):
    pi = pl.program_id(0)

    @pl.when(pi == 0)
    def _():
        pool = jnp.max(pool_ref[...], axis=0)
        v = jnp.dot(pool, vw1_ref[...], preferred_element_type=_f32) + vb1_ref[...]
        v = _ln(v, vlng_ref[...], vlnb_ref[...])
        v = jnp.dot(_gelu(v), vw2_ref[...], preferred_element_type=_f32) + vb2_ref[...]
        vn_new = vn_ref[...] + v
        vn_scr[...] = vn_new
        vnn_ref[...] = vn_new
        cc = c_ref[...]
        sc = cc * jax.nn.sigmoid(cc)
        ssg_scr[...] = jnp.dot(sc, aw_ref[...],
                               preferred_element_type=_f32) + ab_ref[...]

    hin = hin_ref[...]
    agg = agg_ref[0] + agg_ref[1]
    oh = oh_ref[...]
    hn = _dense_hn(False, hin, agg, oh, ssg_scr[...], eps_ref[0, 0],
                   w1_ref[...], b1_ref[...], lng_ref[...], lnb_ref[...],
                   w2_ref[...], b2_ref[...])
    hnext = hn + jnp.dot(oh, vn_scr[...], preferred_element_type=_f32)
    hinn_ref[...] = hnext
    for bb in range(NBOND):
        gn_ref[bb] = _gelu(hnext + bondn_ref[bb])


def _tcmid_stage(hin, agg, pool, oh, c, aw, ab, eps, w1, b1, lng, lnb, w2,
                 b2, vw1, vb1, vlng, vlnb, vw2, vb2, vn, bondn):
    full = lambda *shape: pl.BlockSpec(shape, lambda i: tuple(0 for _ in shape))
    return pl.pallas_call(
        _tcmid_body,
        grid=(NTILES,),
        in_specs=[
            pl.BlockSpec((TILE, H), lambda i: (i, 0)),
            pl.BlockSpec((NC, TILE, H), lambda i: (0, i, 0)),
            full(NW, B, H),
            pl.BlockSpec((TILE, B), lambda i: (i, 0)),
            full(B, TXT),
            full(TXT, 3 * H),
            full(1, 3 * H),
            full(1, 1),
            full(H, 4 * H),
            full(1, 4 * H),
            full(1, 4 * H),
            full(1, 4 * H),
            full(4 * H, H),
            full(1, H),
            full(H, 4 * H),
            full(1, 4 * H),
            full(1, 4 * H),
            full(1, 4 * H),
            full(4 * H, H),
            full(1, H),
            full(B, H),
            pl.BlockSpec((NBOND, 1, H), lambda i: (0, 0, 0)),
        ],
        out_specs=[
            pl.BlockSpec((TILE, H), lambda i: (i, 0)),
            pl.BlockSpec((NBOND, TILE, H), lambda i: (0, i, 0)),
            full(B, H),
        ],
        out_shape=[
            jax.ShapeDtypeStruct((NN, H), _f32),
            jax.ShapeDtypeStruct((NBOND, NN, H), _f32),
            jax.ShapeDtypeStruct((B, H), _f32),
        ],
        scratch_shapes=[
            pltpu.VMEM((B, H), _f32),
            pltpu.VMEM((B, 3 * H), _f32),
        ],
    )(hin, agg, pool, oh, c, aw, ab, eps, w1, b1, lng, lnb, w2, b2,
      vw1, vb1, vlng, vlnb, vw2, vb2, vn, bondn)


def _tcfin_body(hin_ref, agg_ref, oh_ref, c_ref, aw_ref, ab_ref,
                eps_ref, w1_ref, b1_ref, lng_ref, lnb_ref, w2_ref, b2_ref,
                dw1_ref, db1_ref, dlng_ref, dlnb_ref, dw2_ref, db2_ref,
                o_ref, ssg_scr, hg_scr):
    pi = pl.program_id(0)

    @pl.when(pi == 0)
    def _():
        cc = c_ref[...]
        sc = cc * jax.nn.sigmoid(cc)
        ssg_scr[...] = jnp.dot(sc, aw_ref[...],
                               preferred_element_type=_f32) + ab_ref[...]
        hg_scr[...] = jnp.zeros((B, H), _f32)

    hin = hin_ref[...]
    agg = agg_ref[0] + agg_ref[1]
    oh = oh_ref[...]
    hn = _dense_hn(True, hin, agg, oh, ssg_scr[...], eps_ref[0, 0],
                   w1_ref[...], b1_ref[...], lng_ref[...], lnb_ref[...],
                   w2_ref[...], b2_ref[...])
    hg_scr[...] += lax.dot_general(oh, hn, (((0,), (0,)), ((), ())),
                                   preferred_element_type=_f32)

    @pl.when(pi == NTILES - 1)
    def _():
        o = jnp.dot(hg_scr[...], dw1_ref[...],
                    preferred_element_type=_f32) + db1_ref[...]
        o = _ln(o, dlng_ref[...], dlnb_ref[...])
        o_ref[...] = jnp.dot(_gelu(o), dw2_ref[...],
                             preferred_element_type=_f32) + db2_ref[...]


def _tcfin_stage(hin, agg, oh, c, aw, ab, eps, w1, b1, lng, lnb, w2, b2,
                 dw1, db1, dlng, dlnb, dw2, db2):
    full = lambda *shape: pl.BlockSpec(shape, lambda i: tuple(0 for _ in shape))
    return pl.pallas_call(
        _tcfin_body,
        grid=(NTILES,),
        in_specs=[
            pl.BlockSpec((TILE, H), lambda i: (i, 0)),
            pl.BlockSpec((NC, TILE, H), lambda i: (0, i, 0)),
            pl.BlockSpec((TILE, B), lambda i: (i, 0)),
            full(B, TXT),
            full(TXT, 3 * H),
            full(1, 3 * H),
            full(1, 1),
            full(H, 4 * H),
            full(1, 4 * H),
            full(1, 4 * H),
            full(1, 4 * H),
            full(4 * H, H),
            full(1, H),
            full(H, 4 * H),
            full(1, 4 * H),
            full(1, 4 * H),
            full(1, 4 * H),
            full(4 * H, OUT),
            full(1, OUT),
        ],
        out_specs=pl.BlockSpec((B, OUT), lambda i: (0, 0)),
        out_shape=jax.ShapeDtypeStruct((B, OUT), _f32),
        scratch_shapes=[
            pltpu.VMEM((B, 3 * H), _f32),
            pltpu.VMEM((B, H), _f32),
        ],
    )(hin, agg, oh, c, aw, ab, eps, w1, b1, lng, lnb, w2, b2,
      dw1, db1, dlng, dlnb, dw2, db2)


def kernel(x, edge_index, edge_attr, batch, c, params):
    p = params
    src = edge_index[0].astype(jnp.int32)
    dst = edge_index[1].astype(jnp.int32)
    ea = edge_attr.astype(jnp.int32)
    batch = batch.astype(jnp.int32)
    x = x.astype(jnp.int32)

    atom_oh = (x[:, None] == jnp.arange(128, dtype=jnp.int32)[None, :]
               ).astype(_f32)
    atom_embp = jnp.concatenate(
        [p['atom_emb'], jnp.zeros((128 - NATOM, H), _f32)], axis=0)
    oh = (batch[:, None] == jnp.arange(B, dtype=jnp.int32)[None, :]
          ).astype(_f32)

    gidx = ea * NN + src
    packed = gidx | (dst << 17)
    padn = NE_PAD - NE
    pad_dst = NN + (jnp.arange(padn, dtype=jnp.int32) % (NACC - NN))
    packed2d = jnp.concatenate(
        [packed, pad_dst << 17]).reshape(NCHT, ECHUNK)

    vn_row = p['vn_emb'].reshape(1, H)
    bond = p['bond_emb'].reshape(L, NBOND, 1, H)
    eps = p['eps'].reshape(L, 1, 1)
    r2 = lambda a: a.reshape(1, -1)

    hin, gtab, vn = _tc0_stage(atom_oh, atom_embp, vn_row, bond[0])

    for l in range(L - 1):
        agg, pool = _sc_edge_stage(packed2d, gtab.reshape(NBOND * NN, H),
                                   hin, batch, True)
        hin, gtab, vn = _tcmid_stage(
            hin, agg, pool, oh, c,
            p['adapt_W'][l], r2(p['adapt_b'][l]), eps[l],
            p['conv_W1'][l], r2(p['conv_b1'][l]),
            r2(p['conv_ln_g'][l]), r2(p['conv_ln_b'][l]),
            p['conv_W2'][l], r2(p['conv_b2'][l]),
            p['vn_W1'][l], r2(p['vn_b1'][l]),
            r2(p['vn_ln_g'][l]), r2(p['vn_ln_b'][l]),
            p['vn_W2'][l], r2(p['vn_b2'][l]),
            vn, bond[l + 1])

    agg, _ = _sc_edge_stage(packed2d, gtab.reshape(NBOND * NN, H),
                            hin, batch, False)
    o = _tcfin_stage(
        hin, agg, oh, c,
        p['adapt_W'][L - 1], r2(p['adapt_b'][L - 1]), eps[L - 1],
        p['conv_W1'][L - 1], r2(p['conv_b1'][L - 1]),
        r2(p['conv_ln_g'][L - 1]), r2(p['conv_ln_b'][L - 1]),
        p['conv_W2'][L - 1], r2(p['conv_b2'][L - 1]),
        p['dec_W1'], r2(p['dec_b1']),
        r2(p['dec_ln_g']), r2(p['dec_ln_b']),
        p['dec_W2'], r2(p['dec_b2']))
    return o


# speedup vs baseline: 1.2283x; 1.2283x over previous
import functools

import jax
import jax.numpy as jnp
from jax import lax
from jax.experimental import pallas as pl
from jax.experimental.pallas import tpu as pltpu
from jax.experimental.pallas import tpu_sc as plsc

L = 5
H = 128
TXT = 768
OUT = 512
NN = 10000
NE = 320000
B = 128
NBOND = 5
NATOM = 118

NC = 2
NS = 16
NW = NC * NS

ECHUNK = 64
NE_PAD = 327680
NCHT = NE_PAD // ECHUNK
ECH0 = 280
ECH1 = 40
NBUF = 5
GDEPTH = 2

NACC = 10112
ROWS_PER_SUB = NACC // NS

NPT = 320

TILE = 1000
NTILES = NN // TILE

_f32 = jnp.float32


_INV_SQRT2 = 0.7071067811865476


def _gelu(x):
    return 0.5 * x * (1.0 + lax.erf(x * _INV_SQRT2))


def _ln(x, g=None, b=None, eps=1e-5):
    mu = jnp.mean(x, axis=-1, keepdims=True)
    var = jnp.var(x, axis=-1, keepdims=True)
    y = (x - mu) / jnp.sqrt(var + eps)
    if g is not None:
        y = y * g + b
    return y


EGRP = 40


def _sc_body(do_pool, pk_hbm, gtab_hbm, hin_hbm, batch_hbm,
             agg_hbm, pool_hbm,
             pk_v, gidx_b, didx_b, buf_v, bat_v, acc_sh,
             gs0, gs1, gs2, gs3, gs4, ss0, ss1, ss2, ss3, ss4):
    c = lax.axis_index("c")
    s = lax.axis_index("s")
    wid = c * NS + s
    gsems = [gs0, gs1, gs2, gs3, gs4]
    ssems = [ss0, ss1, ss2, ss3, ss4]
    rowb = [buf_v.at[pl.ds(u * ECHUNK, ECHUNK)] for u in range(NBUF)]
    part = buf_v.at[pl.ds(0, B)]
    hrow = buf_v.at[pl.ds(128, 64)]

    def _zrow(i, _):
        for k in range(8):
            buf_v[i, pl.ds(k * 16, 16)] = jnp.zeros((16,), _f32)
        return 0
    lax.fori_loop(0, NPT, _zrow, 0)
    pltpu.sync_copy(buf_v.at[pl.ds(0, NPT)],
                    acc_sh.at[pl.ds(s * ROWS_PER_SUB, NPT)])
    pltpu.sync_copy(buf_v.at[pl.ds(0, ROWS_PER_SUB - NPT)],
                    acc_sh.at[pl.ds(s * ROWS_PER_SUB + NPT,
                                    ROWS_PER_SUB - NPT)])
    plsc.subcore_barrier()

    if do_pool:
        base = jnp.minimum(wid * NPT, NN - NPT)
        pltpu.sync_copy(batch_hbm.at[pl.ds(base, NPT)], bat_v.at[pl.ds(0, NPT)])

        def _pinit(i, _):
            for k in range(8):
                part[i, pl.ds(k * 16, 16)] = jnp.full((16,), -1e30, _f32)
            return 0
        lax.fori_loop(0, B, _pinit, 0)

        def _pchunk(cc, _):
            pltpu.sync_copy(hin_hbm.at[pl.ds(base + cc * 64, 64)], hrow)

            def _prow(i, _):
                g = bat_v[pl.ds(cc * 64 + i, 16)][0]
                for k in range(8):
                    sl = pl.ds(k * 16, 16)
                    part[g, sl] = jnp.maximum(part[g, sl], hrow[i, sl])
                return 0
            lax.fori_loop(0, 64, _prow, 0)
            return 0
        lax.fori_loop(0, NPT // 64, _pchunk, 0)
        pltpu.sync_copy(part, pool_hbm.at[wid])

    cbase = jnp.where(c == 0, s * ECH0, NS * ECH0 + s * ECH1)
    ngrp = jnp.where(c == 0, ECH0 // EGRP, ECH1 // EGRP)

    def _unpack(lj, u):
        for k in range(ECHUNK // 16):
            sl = pl.ds(k * 16, 16)
            v = pk_v[lj, sl]
            gidx_b[u, sl] = v & 0x1FFFF
            didx_b[u, sl] = v >> 17

    def _egrp(grp, _):
        off = cbase + grp * EGRP
        pltpu.sync_copy(pk_hbm.at[pl.ds(off, EGRP)], pk_v)
        for u in range(GDEPTH):
            _unpack(u, u)
            pltpu.async_copy(gtab_hbm.at[gidx_b.at[u]], rowb[u], gsems[u])

        def _emac(m, _):
            for u in range(NBUF):
                lj = m * NBUF + u
                pltpu.make_async_copy(gtab_hbm.at[gidx_b.at[u]], rowb[u],
                                      gsems[u]).wait()
                pltpu.async_copy(rowb[u], acc_sh.at[didx_b.at[u]],
                                 ssems[u], add=True)
                bt = (u + GDEPTH) % NBUF
                lt = lj + GDEPTH

                @pl.when(lt < EGRP)
                def _():
                    @pl.when(lt >= NBUF)
                    def _():
                        pltpu.make_async_copy(
                            rowb[bt], acc_sh.at[didx_b.at[bt]],
                            ssems[bt]).wait()
                    _unpack(lt, bt)
                    pltpu.async_copy(gtab_hbm.at[gidx_b.at[bt]], rowb[bt],
                                     gsems[bt])
            return 0

        lax.fori_loop(0, EGRP // NBUF, _emac, 0)
        for u in range(NBUF):
            pltpu.make_async_copy(rowb[u], acc_sh.at[didx_b.at[u]],
                                  ssems[u]).wait()
        return 0

    lax.fori_loop(0, ngrp, _egrp, 0)
    plsc.subcore_barrier()

    pltpu.sync_copy(acc_sh.at[pl.ds(s * ROWS_PER_SUB, ROWS_PER_SUB)],
                    agg_hbm.at[c, pl.ds(s * ROWS_PER_SUB, ROWS_PER_SUB)])


def _sc_edge_stage(packed2d, gtab, hin, batch, do_pool):
    mesh = plsc.VectorSubcoreMesh(core_axis_name="c", subcore_axis_name="s")
    fn = pl.kernel(
        functools.partial(_sc_body, do_pool),
        out_type=[
            jax.ShapeDtypeStruct((NC, NACC, H), _f32),
            jax.ShapeDtypeStruct((NW, B, H), _f32),
        ],
        mesh=mesh,
        scratch_types=(
            [
                pltpu.VMEM((EGRP, ECHUNK), jnp.int32),
                pltpu.VMEM((NBUF, ECHUNK), jnp.int32),
                pltpu.VMEM((NBUF, ECHUNK), jnp.int32),
                pltpu.VMEM((NBUF * ECHUNK, H), _f32),
                pltpu.VMEM((NPT + 16,), jnp.int32),
                pltpu.VMEM_SHARED((NACC, H), _f32),
            ]
            + [pltpu.SemaphoreType.DMA] * (2 * NBUF)
        ),
    )
    return fn(packed2d, gtab, hin, batch)


def _tc0_body(aoh_ref, aemb_ref, vnrow_ref, bond_ref,
              hin_ref, g_ref, vn0_ref):
    pi = pl.program_id(0)
    h = jnp.dot(aoh_ref[...], aemb_ref[...],
                preferred_element_type=_f32) + vnrow_ref[...]
    hin_ref[...] = h
    for bb in range(NBOND):
        g_ref[bb] = _gelu(h + bond_ref[bb])

    @pl.when(pi == 0)
    def _():
        vn0_ref[...] = jnp.broadcast_to(vnrow_ref[...], (B, H))


def _tc0_stage(atom_oh, atom_embp, vn_row, bond0):
    return pl.pallas_call(
        _tc0_body,
        grid=(NTILES,),
        in_specs=[
            pl.BlockSpec((TILE, 128), lambda i: (i, 0)),
            pl.BlockSpec((128, H), lambda i: (0, 0)),
            pl.BlockSpec((1, H), lambda i: (0, 0)),
            pl.BlockSpec((NBOND, 1, H), lambda i: (0, 0, 0)),
        ],
        out_specs=[
            pl.BlockSpec((TILE, H), lambda i: (i, 0)),
            pl.BlockSpec((NBOND, TILE, H), lambda i: (0, i, 0)),
            pl.BlockSpec((B, H), lambda i: (0, 0)),
        ],
        out_shape=[
            jax.ShapeDtypeStruct((NN, H), _f32),
            jax.ShapeDtypeStruct((NBOND, NN, H), _f32),
            jax.ShapeDtypeStruct((B, H), _f32),
        ],
    )(atom_oh, atom_embp, vn_row, bond0)


def _dense_hn(last, hin, agg, oh, ssg, eps, w1, b1, lng, lnb, w2, b2):
    z = (1.0 + eps) * hin + agg
    z = jnp.dot(z, w1, preferred_element_type=_f32) + b1
    z = _ln(z, lng, lnb)
    z = jnp.dot(_gelu(z), w2, preferred_element_type=_f32) + b2
    hn = _ln(z)
    mod = jnp.dot(oh, ssg, preferred_element_type=_f32)
    shift = mod[:, :H]
    scale = mod[:, H:2 * H]
    gate = mod[:, 2 * H:]
    hn = hn * (1.0 + scale) + shift
    if not last:
        hn = _gelu(hn)
    return gate * hn + hin


def _tcmid_body(hin_ref, agg_ref, pool_ref, oh_ref, c_ref, aw_ref, ab_ref,
                eps_ref, w1_ref, b1_ref, lng_ref, lnb_ref, w2_ref, b2_ref,
                vw1_ref, vb1_ref, vlng_ref, vlnb_ref, vw2_ref, vb2_ref,
                vn_ref, bondn_ref,
                hinn_ref, gn_ref, vnn_ref,
                vn_scr, ssg_scr):
    pi = pl.program_id(0)

    @pl.when(pi == 0)
    def _():
        pool = jnp.max(pool_ref[...], axis=0)
        v = jnp.dot(pool, vw1_ref[...], preferred_element_type=_f32) + vb1_ref[...]
        v = _ln(v, vlng_ref[...], vlnb_ref[...])
        v = jnp.dot(_gelu(v), vw2_ref[...], preferred_element_type=_f32) + vb2_ref[...]
        vn_new = vn_ref[...] + v
        vn_scr[...] = vn_new
        vnn_ref[...] = vn_new
        cc = c_ref[...]
        sc = cc * jax.nn.sigmoid(cc)
        ssg_scr[...] = jnp.dot(sc, aw_ref[...],
                               preferred_element_type=_f32) + ab_ref[...]

    hin = hin_ref[...]
    agg = agg_ref[0] + agg_ref[1]
    oh = oh_ref[...]
    hn = _dense_hn(False, hin, agg, oh, ssg_scr[...], eps_ref[0, 0],
                   w1_ref[...], b1_ref[...], lng_ref[...], lnb_ref[...],
                   w2_ref[...], b2_ref[...])
    hnext = hn + jnp.dot(oh, vn_scr[...], preferred_element_type=_f32)
    hinn_ref[...] = hnext
    for bb in range(NBOND):
        gn_ref[bb] = _gelu(hnext + bondn_ref[bb])


def _tcmid_stage(hin, agg, pool, oh, c, aw, ab, eps, w1, b1, lng, lnb, w2,
                 b2, vw1, vb1, vlng, vlnb, vw2, vb2, vn, bondn):
    full = lambda *shape: pl.BlockSpec(shape, lambda i: tuple(0 for _ in shape))
    return pl.pallas_call(
        _tcmid_body,
        grid=(NTILES,),
        in_specs=[
            pl.BlockSpec((TILE, H), lambda i: (i, 0)),
            pl.BlockSpec((NC, TILE, H), lambda i: (0, i, 0)),
            full(NW, B, H),
            pl.BlockSpec((TILE, B), lambda i: (i, 0)),
            full(B, TXT),
            full(TXT, 3 * H),
            full(1, 3 * H),
            full(1, 1),
            full(H, 4 * H),
            full(1, 4 * H),
            full(1, 4 * H),
            full(1, 4 * H),
            full(4 * H, H),
            full(1, H),
            full(H, 4 * H),
            full(1, 4 * H),
            full(1, 4 * H),
            full(1, 4 * H),
            full(4 * H, H),
            full(1, H),
            full(B, H),
            pl.BlockSpec((NBOND, 1, H), lambda i: (0, 0, 0)),
        ],
        out_specs=[
            pl.BlockSpec((TILE, H), lambda i: (i, 0)),
            pl.BlockSpec((NBOND, TILE, H), lambda i: (0, i, 0)),
            full(B, H),
        ],
        out_shape=[
            jax.ShapeDtypeStruct((NN, H), _f32),
            jax.ShapeDtypeStruct((NBOND, NN, H), _f32),
            jax.ShapeDtypeStruct((B, H), _f32),
        ],
        scratch_shapes=[
            pltpu.VMEM((B, H), _f32),
            pltpu.VMEM((B, 3 * H), _f32),
        ],
    )(hin, agg, pool, oh, c, aw, ab, eps, w1, b1, lng, lnb, w2, b2,
      vw1, vb1, vlng, vlnb, vw2, vb2, vn, bondn)


def _tcfin_body(hin_ref, agg_ref, oh_ref, c_ref, aw_ref, ab_ref,
                eps_ref, w1_ref, b1_ref, lng_ref, lnb_ref, w2_ref, b2_ref,
                dw1_ref, db1_ref, dlng_ref, dlnb_ref, dw2_ref, db2_ref,
                o_ref, ssg_scr, hg_scr):
    pi = pl.program_id(0)

    @pl.when(pi == 0)
    def _():
        cc = c_ref[...]
        sc = cc * jax.nn.sigmoid(cc)
        ssg_scr[...] = jnp.dot(sc, aw_ref[...],
                               preferred_element_type=_f32) + ab_ref[...]
        hg_scr[...] = jnp.zeros((B, H), _f32)

    hin = hin_ref[...]
    agg = agg_ref[0] + agg_ref[1]
    oh = oh_ref[...]
    hn = _dense_hn(True, hin, agg, oh, ssg_scr[...], eps_ref[0, 0],
                   w1_ref[...], b1_ref[...], lng_ref[...], lnb_ref[...],
                   w2_ref[...], b2_ref[...])
    hg_scr[...] += lax.dot_general(oh, hn, (((0,), (0,)), ((), ())),
                                   preferred_element_type=_f32)

    @pl.when(pi == NTILES - 1)
    def _():
        o = jnp.dot(hg_scr[...], dw1_ref[...],
                    preferred_element_type=_f32) + db1_ref[...]
        o = _ln(o, dlng_ref[...], dlnb_ref[...])
        o_ref[...] = jnp.dot(_gelu(o), dw2_ref[...],
                             preferred_element_type=_f32) + db2_ref[...]


def _tcfin_stage(hin, agg, oh, c, aw, ab, eps, w1, b1, lng, lnb, w2, b2,
                 dw1, db1, dlng, dlnb, dw2, db2):
    full = lambda *shape: pl.BlockSpec(shape, lambda i: tuple(0 for _ in shape))
    return pl.pallas_call(
        _tcfin_body,
        grid=(NTILES,),
        in_specs=[
            pl.BlockSpec((TILE, H), lambda i: (i, 0)),
            pl.BlockSpec((NC, TILE, H), lambda i: (0, i, 0)),
            pl.BlockSpec((TILE, B), lambda i: (i, 0)),
            full(B, TXT),
            full(TXT, 3 * H),
            full(1, 3 * H),
            full(1, 1),
            full(H, 4 * H),
            full(1, 4 * H),
            full(1, 4 * H),
            full(1, 4 * H),
            full(4 * H, H),
            full(1, H),
            full(H, 4 * H),
            full(1, 4 * H),
            full(1, 4 * H),
            full(1, 4 * H),
            full(4 * H, OUT),
            full(1, OUT),
        ],
        out_specs=pl.BlockSpec((B, OUT), lambda i: (0, 0)),
        out_shape=jax.ShapeDtypeStruct((B, OUT), _f32),
        scratch_shapes=[
            pltpu.VMEM((B, 3 * H), _f32),
            pltpu.VMEM((B, H), _f32),
        ],
    )(hin, agg, oh, c, aw, ab, eps, w1, b1, lng, lnb, w2, b2,
      dw1, db1, dlng, dlnb, dw2, db2)


def kernel(x, edge_index, edge_attr, batch, c, params):
    p = params
    src = edge_index[0].astype(jnp.int32)
    dst = edge_index[1].astype(jnp.int32)
    ea = edge_attr.astype(jnp.int32)
    batch = batch.astype(jnp.int32)
    x = x.astype(jnp.int32)

    atom_oh = (x[:, None] == jnp.arange(128, dtype=jnp.int32)[None, :]
               ).astype(_f32)
    atom_embp = jnp.concatenate(
        [p['atom_emb'], jnp.zeros((128 - NATOM, H), _f32)], axis=0)
    oh = (batch[:, None] == jnp.arange(B, dtype=jnp.int32)[None, :]
          ).astype(_f32)

    gidx = ea * NN + src
    packed = gidx | (dst << 17)
    padn = NE_PAD - NE
    pad_dst = NN + (jnp.arange(padn, dtype=jnp.int32) % (NACC - NN))
    packed2d = jnp.concatenate(
        [packed, pad_dst << 17]).reshape(NCHT, ECHUNK)

    vn_row = p['vn_emb'].reshape(1, H)
    bond = p['bond_emb'].reshape(L, NBOND, 1, H)
    eps = p['eps'].reshape(L, 1, 1)
    r2 = lambda a: a.reshape(1, -1)

    hin, gtab, vn = _tc0_stage(atom_oh, atom_embp, vn_row, bond[0])

    for l in range(L - 1):
        agg, pool = _sc_edge_stage(packed2d, gtab.reshape(NBOND * NN, H),
                                   hin, batch, True)
        hin, gtab, vn = _tcmid_stage(
            hin, agg, pool, oh, c,
            p['adapt_W'][l], r2(p['adapt_b'][l]), eps[l],
            p['conv_W1'][l], r2(p['conv_b1'][l]),
            r2(p['conv_ln_g'][l]), r2(p['conv_ln_b'][l]),
            p['conv_W2'][l], r2(p['conv_b2'][l]),
            p['vn_W1'][l], r2(p['vn_b1'][l]),
            r2(p['vn_ln_g'][l]), r2(p['vn_ln_b'][l]),
            p['vn_W2'][l], r2(p['vn_b2'][l]),
            vn, bond[l + 1])

    agg, _ = _sc_edge_stage(packed2d, gtab.reshape(NBOND * NN, H),
                            hin, batch, False)
    o = _tcfin_stage(
        hin, agg, oh, c,
        p['adapt_W'][L - 1], r2(p['adapt_b'][L - 1]), eps[L - 1],
        p['conv_W1'][L - 1], r2(p['conv_b1'][L - 1]),
        r2(p['conv_ln_g'][L - 1]), r2(p['conv_ln_b'][L - 1]),
        p['conv_W2'][L - 1], r2(p['conv_b2'][L - 1]),
        p['dec_W1'], r2(p['dec_b1']),
        r2(p['dec_ln_g']), r2(p['dec_ln_b']),
        p['dec_W2'], r2(p['dec_b2']))
    return o


# speedup vs baseline: 1.2504x; 1.0180x over previous
import functools

import jax
import jax.numpy as jnp
from jax import lax
from jax.experimental import pallas as pl
from jax.experimental.pallas import tpu as pltpu
from jax.experimental.pallas import tpu_sc as plsc

L = 5
H = 128
TXT = 768
OUT = 512
NN = 10000
NE = 320000
B = 128
NBOND = 5
NATOM = 118

NC = 2
NS = 16
NW = NC * NS

ECHUNK = 64
NE_PAD = 327680
NCHT = NE_PAD // ECHUNK
ECH0 = 280
ECH1 = 40
NBUF = 5
GDEPTH = 3

NACC = 10112
ROWS_PER_SUB = NACC // NS

NPT = 320

TILE = 1000
NTILES = NN // TILE

_f32 = jnp.float32


_INV_SQRT2 = 0.7071067811865476


def _gelu(x):
    return 0.5 * x * (1.0 + lax.erf(x * _INV_SQRT2))


def _ln(x, g=None, b=None, eps=1e-5):
    mu = jnp.mean(x, axis=-1, keepdims=True)
    var = jnp.var(x, axis=-1, keepdims=True)
    y = (x - mu) / jnp.sqrt(var + eps)
    if g is not None:
        y = y * g + b
    return y


EGRP = 40


def _sc_body(do_pool, pk_hbm, gtab_hbm, hin_hbm, batch_hbm,
             agg_hbm, pool_hbm,
             pk_v, gidx_b, didx_b, buf_v, bat_v, acc_sh,
             gs0, gs1, gs2, gs3, gs4, ss0, ss1, ss2, ss3, ss4):
    c = lax.axis_index("c")
    s = lax.axis_index("s")
    wid = c * NS + s
    gsems = [gs0, gs1, gs2, gs3, gs4]
    ssems = [ss0, ss1, ss2, ss3, ss4]
    rowb = [buf_v.at[pl.ds(u * ECHUNK, ECHUNK)] for u in range(NBUF)]
    part = buf_v.at[pl.ds(0, B)]
    hrow = buf_v.at[pl.ds(128, 64)]

    def _zrow(i, _):
        for k in range(8):
            buf_v[i, pl.ds(k * 16, 16)] = jnp.zeros((16,), _f32)
        return 0
    lax.fori_loop(0, NPT, _zrow, 0)
    pltpu.sync_copy(buf_v.at[pl.ds(0, NPT)],
                    acc_sh.at[pl.ds(s * ROWS_PER_SUB, NPT)])
    pltpu.sync_copy(buf_v.at[pl.ds(0, ROWS_PER_SUB - NPT)],
                    acc_sh.at[pl.ds(s * ROWS_PER_SUB + NPT,
                                    ROWS_PER_SUB - NPT)])
    plsc.subcore_barrier()

    if do_pool:
        base = jnp.minimum(wid * NPT, NN - NPT)
        pltpu.sync_copy(batch_hbm.at[pl.ds(base, NPT)], bat_v.at[pl.ds(0, NPT)])

        def _pinit(i, _):
            for k in range(8):
                part[i, pl.ds(k * 16, 16)] = jnp.full((16,), -1e30, _f32)
            return 0
        lax.fori_loop(0, B, _pinit, 0)

        def _pchunk(cc, _):
            pltpu.sync_copy(hin_hbm.at[pl.ds(base + cc * 64, 64)], hrow)

            def _prow(i, _):
                g = bat_v[pl.ds(cc * 64 + i, 16)][0]
                for k in range(8):
                    sl = pl.ds(k * 16, 16)
                    part[g, sl] = jnp.maximum(part[g, sl], hrow[i, sl])
                return 0
            lax.fori_loop(0, 64, _prow, 0)
            return 0
        lax.fori_loop(0, NPT // 64, _pchunk, 0)
        pltpu.sync_copy(part, pool_hbm.at[wid])

    cbase = jnp.where(c == 1, s * ECH0, NS * ECH0 + s * ECH1)
    ngrp = jnp.where(c == 1, ECH0 // EGRP, ECH1 // EGRP)

    def _unpack(lj, u):
        for k in range(ECHUNK // 16):
            sl = pl.ds(k * 16, 16)
            v = pk_v[lj, sl]
            gidx_b[u, sl] = v & 0x1FFFF
            didx_b[u, sl] = v >> 17

    def _egrp(grp, _):
        off = cbase + grp * EGRP
        pltpu.sync_copy(pk_hbm.at[pl.ds(off, EGRP)], pk_v)
        for u in range(GDEPTH):
            _unpack(u, u)
            pltpu.async_copy(gtab_hbm.at[gidx_b.at[u]], rowb[u], gsems[u])

        def _emac(m, _):
            for u in range(NBUF):
                lj = m * NBUF + u
                pltpu.make_async_copy(gtab_hbm.at[gidx_b.at[u]], rowb[u],
                                      gsems[u]).wait()
                pltpu.async_copy(rowb[u], acc_sh.at[didx_b.at[u]],
                                 ssems[u], add=True)
                bt = (u + GDEPTH) % NBUF
                lt = lj + GDEPTH

                @pl.when(lt < EGRP)
                def _():
                    @pl.when(lt >= NBUF)
                    def _():
                        pltpu.make_async_copy(
                            rowb[bt], acc_sh.at[didx_b.at[bt]],
                            ssems[bt]).wait()
                    _unpack(lt, bt)
                    pltpu.async_copy(gtab_hbm.at[gidx_b.at[bt]], rowb[bt],
                                     gsems[bt])
            return 0

        lax.fori_loop(0, EGRP // NBUF, _emac, 0)
        for u in range(NBUF):
            pltpu.make_async_copy(rowb[u], acc_sh.at[didx_b.at[u]],
                                  ssems[u]).wait()
        return 0

    lax.fori_loop(0, ngrp, _egrp, 0)
    plsc.subcore_barrier()

    pltpu.sync_copy(acc_sh.at[pl.ds(s * ROWS_PER_SUB, ROWS_PER_SUB)],
                    agg_hbm.at[c, pl.ds(s * ROWS_PER_SUB, ROWS_PER_SUB)])


def _sc_edge_stage(packed2d, gtab, hin, batch, do_pool):
    mesh = plsc.VectorSubcoreMesh(core_axis_name="c", subcore_axis_name="s")
    fn = pl.kernel(
        functools.partial(_sc_body, do_pool),
        out_type=[
            jax.ShapeDtypeStruct((NC, NACC, H), _f32),
            jax.ShapeDtypeStruct((NW, B, H), _f32),
        ],
        mesh=mesh,
        scratch_types=(
            [
                pltpu.VMEM((EGRP, ECHUNK), jnp.int32),
                pltpu.VMEM((NBUF, ECHUNK), jnp.int32),
                pltpu.VMEM((NBUF, ECHUNK), jnp.int32),
                pltpu.VMEM((NBUF * ECHUNK, H), _f32),
                pltpu.VMEM((NPT + 16,), jnp.int32),
                pltpu.VMEM_SHARED((NACC, H), _f32),
            ]
            + [pltpu.SemaphoreType.DMA] * (2 * NBUF)
        ),
    )
    return fn(packed2d, gtab, hin, batch)


def _tc0_body(aoh_ref, aemb_ref, vnrow_ref, bond_ref,
              hin_ref, g_ref, vn0_ref):
    pi = pl.program_id(0)
    h = jnp.dot(aoh_ref[...], aemb_ref[...],
                preferred_element_type=_f32) + vnrow_ref[...]
    hin_ref[...] = h
    for bb in range(NBOND):
        g_ref[bb] = _gelu(h + bond_ref[bb])

    @pl.when(pi == 0)
    def _():
        vn0_ref[...] = jnp.broadcast_to(vnrow_ref[...], (B, H))


def _tc0_stage(atom_oh, atom_embp, vn_row, bond0):
    return pl.pallas_call(
        _tc0_body,
        grid=(NTILES,),
        in_specs=[
            pl.BlockSpec((TILE, 128), lambda i: (i, 0)),
            pl.BlockSpec((128, H), lambda i: (0, 0)),
            pl.BlockSpec((1, H), lambda i: (0, 0)),
            pl.BlockSpec((NBOND, 1, H), lambda i: (0, 0, 0)),
        ],
        out_specs=[
            pl.BlockSpec((TILE, H), lambda i: (i, 0)),
            pl.BlockSpec((NBOND, TILE, H), lambda i: (0, i, 0)),
            pl.BlockSpec((B, H), lambda i: (0, 0)),
        ],
        out_shape=[
            jax.ShapeDtypeStruct((NN, H), _f32),
            jax.ShapeDtypeStruct((NBOND, NN, H), _f32),
            jax.ShapeDtypeStruct((B, H), _f32),
        ],
    )(atom_oh, atom_embp, vn_row, bond0)


def _dense_hn(last, hin, agg, oh, ssg, eps, w1, b1, lng, lnb, w2, b2):
    z = (1.0 + eps) * hin + agg
    z = jnp.dot(z, w1, preferred_element_type=_f32) + b1
    z = _ln(z, lng, lnb)
    z = jnp.dot(_gelu(z), w2, preferred_element_type=_f32) + b2
    hn = _ln(z)
    mod = jnp.dot(oh, ssg, preferred_element_type=_f32)
    shift = mod[:, :H]
    scale = mod[:, H:2 * H]
    gate = mod[:, 2 * H:]
    hn = hn * (1.0 + scale) + shift
    if not last:
        hn = _gelu(hn)
    return gate * hn + hin


def _tcmid_body(hin_ref, agg_ref, pool_ref, oh_ref, c_ref, aw_ref, ab_ref,
                eps_ref, w1_ref, b1_ref, lng_ref, lnb_ref, w2_ref, b2_ref,
                vw1_ref, vb1_ref, vlng_ref, vlnb_ref, vw2_ref, vb2_ref,
                vn_ref, bondn_ref,
                hinn_ref, gn_ref, vnn_ref,
                vn_scr, ssg_scr):
    pi = pl.program_id(0)

    @pl.when(pi == 0)
    def _():
        pool = jnp.max(pool_ref[...], axis=0)
        v = jnp.dot(pool, vw1_ref[...], preferred_element_type=_f32) + vb1_ref[...]
        v = _ln(v, vlng_ref[...], vlnb_ref[...])
        v = jnp.dot(_gelu(v), vw2_ref[...], preferred_element_type=_f32) + vb2_ref[...]
        vn_new = vn_ref[...] + v
        vn_scr[...] = vn_new
        vnn_ref[...] = vn_new
        cc = c_ref[...]
        sc = cc * jax.nn.sigmoid(cc)
        ssg_scr[...] = jnp.dot(sc, aw_ref[...],
                               preferred_element_type=_f32) + ab_ref[...]

    hin = hin_ref[...]
    agg = agg_ref[0] + agg_ref[1]
    oh = oh_ref[...]
    hn = _dense_hn(False, hin, agg, oh, ssg_scr[...], eps_ref[0, 0],
                   w1_ref[...], b1_ref[...], lng_ref[...], lnb_ref[...],
                   w2_ref[...], b2_ref[...])
    hnext = hn + jnp.dot(oh, vn_scr[...], preferred_element_type=_f32)
    hinn_ref[...] = hnext
    for bb in range(NBOND):
        gn_ref[bb] = _gelu(hnext + bondn_ref[bb])


def _tcmid_stage(hin, agg, pool, oh, c, aw, ab, eps, w1, b1, lng, lnb, w2,
                 b2, vw1, vb1, vlng, vlnb, vw2, vb2, vn, bondn):
    full = lambda *shape: pl.BlockSpec(shape, lambda i: tuple(0 for _ in shape))
    return pl.pallas_call(
        _tcmid_body,
        grid=(NTILES,),
        in_specs=[
            pl.BlockSpec((TILE, H), lambda i: (i, 0)),
            pl.BlockSpec((NC, TILE, H), lambda i: (0, i, 0)),
            full(NW, B, H),
            pl.BlockSpec((TILE, B), lambda i: (i, 0)),
            full(B, TXT),
            full(TXT, 3 * H),
            full(1, 3 * H),
            full(1, 1),
            full(H, 4 * H),
            full(1, 4 * H),
            full(1, 4 * H),
            full(1, 4 * H),
            full(4 * H, H),
            full(1, H),
            full(H, 4 * H),
            full(1, 4 * H),
            full(1, 4 * H),
            full(1, 4 * H),
            full(4 * H, H),
            full(1, H),
            full(B, H),
            pl.BlockSpec((NBOND, 1, H), lambda i: (0, 0, 0)),
        ],
        out_specs=[
            pl.BlockSpec((TILE, H), lambda i: (i, 0)),
            pl.BlockSpec((NBOND, TILE, H), lambda i: (0, i, 0)),
            full(B, H),
        ],
        out_shape=[
            jax.ShapeDtypeStruct((NN, H), _f32),
            jax.ShapeDtypeStruct((NBOND, NN, H), _f32),
            jax.ShapeDtypeStruct((B, H), _f32),
        ],
        scratch_shapes=[
            pltpu.VMEM((B, H), _f32),
            pltpu.VMEM((B, 3 * H), _f32),
        ],
    )(hin, agg, pool, oh, c, aw, ab, eps, w1, b1, lng, lnb, w2, b2,
      vw1, vb1, vlng, vlnb, vw2, vb2, vn, bondn)


def _tcfin_body(hin_ref, agg_ref, oh_ref, c_ref, aw_ref, ab_ref,
                eps_ref, w1_ref, b1_ref, lng_ref, lnb_ref, w2_ref, b2_ref,
                dw1_ref, db1_ref, dlng_ref, dlnb_ref, dw2_ref, db2_ref,
                o_ref, ssg_scr, hg_scr):
    pi = pl.program_id(0)

    @pl.when(pi == 0)
    def _():
        cc = c_ref[...]
        sc = cc * jax.nn.sigmoid(cc)
        ssg_scr[...] = jnp.dot(sc, aw_ref[...],
                               preferred_element_type=_f32) + ab_ref[...]
        hg_scr[...] = jnp.zeros((B, H), _f32)

    hin = hin_ref[...]
    agg = agg_ref[0] + agg_ref[1]
    oh = oh_ref[...]
    hn = _dense_hn(True, hin, agg, oh, ssg_scr[...], eps_ref[0, 0],
                   w1_ref[...], b1_ref[...], lng_ref[...], lnb_ref[...],
                   w2_ref[...], b2_ref[...])
    hg_scr[...] += lax.dot_general(oh, hn, (((0,), (0,)), ((), ())),
                                   preferred_element_type=_f32)

    @pl.when(pi == NTILES - 1)
    def _():
        o = jnp.dot(hg_scr[...], dw1_ref[...],
                    preferred_element_type=_f32) + db1_ref[...]
        o = _ln(o, dlng_ref[...], dlnb_ref[...])
        o_ref[...] = jnp.dot(_gelu(o), dw2_ref[...],
                             preferred_element_type=_f32) + db2_ref[...]


def _tcfin_stage(hin, agg, oh, c, aw, ab, eps, w1, b1, lng, lnb, w2, b2,
                 dw1, db1, dlng, dlnb, dw2, db2):
    full = lambda *shape: pl.BlockSpec(shape, lambda i: tuple(0 for _ in shape))
    return pl.pallas_call(
        _tcfin_body,
        grid=(NTILES,),
        in_specs=[
            pl.BlockSpec((TILE, H), lambda i: (i, 0)),
            pl.BlockSpec((NC, TILE, H), lambda i: (0, i, 0)),
            pl.BlockSpec((TILE, B), lambda i: (i, 0)),
            full(B, TXT),
            full(TXT, 3 * H),
            full(1, 3 * H),
            full(1, 1),
            full(H, 4 * H),
            full(1, 4 * H),
            full(1, 4 * H),
            full(1, 4 * H),
            full(4 * H, H),
            full(1, H),
            full(H, 4 * H),
            full(1, 4 * H),
            full(1, 4 * H),
            full(1, 4 * H),
            full(4 * H, OUT),
            full(1, OUT),
        ],
        out_specs=pl.BlockSpec((B, OUT), lambda i: (0, 0)),
        out_shape=jax.ShapeDtypeStruct((B, OUT), _f32),
        scratch_shapes=[
            pltpu.VMEM((B, 3 * H), _f32),
            pltpu.VMEM((B, H), _f32),
        ],
    )(hin, agg, oh, c, aw, ab, eps, w1, b1, lng, lnb, w2, b2,
      dw1, db1, dlng, dlnb, dw2, db2)


def kernel(x, edge_index, edge_attr, batch, c, params):
    p = params
    src = edge_index[0].astype(jnp.int32)
    dst = edge_index[1].astype(jnp.int32)
    ea = edge_attr.astype(jnp.int32)
    batch = batch.astype(jnp.int32)
    x = x.astype(jnp.int32)

    atom_oh = (x[:, None] == jnp.arange(128, dtype=jnp.int32)[None, :]
               ).astype(_f32)
    atom_embp = jnp.concatenate(
        [p['atom_emb'], jnp.zeros((128 - NATOM, H), _f32)], axis=0)
    oh = (batch[:, None] == jnp.arange(B, dtype=jnp.int32)[None, :]
          ).astype(_f32)

    gidx = ea * NN + src
    packed = gidx | (dst << 17)
    padn = NE_PAD - NE
    pad_dst = NN + (jnp.arange(padn, dtype=jnp.int32) % (NACC - NN))
    packed2d = jnp.concatenate(
        [packed, pad_dst << 17]).reshape(NCHT, ECHUNK)

    vn_row = p['vn_emb'].reshape(1, H)
    bond = p['bond_emb'].reshape(L, NBOND, 1, H)
    eps = p['eps'].reshape(L, 1, 1)
    r2 = lambda a: a.reshape(1, -1)

    hin, gtab, vn = _tc0_stage(atom_oh, atom_embp, vn_row, bond[0])

    for l in range(L - 1):
        agg, pool = _sc_edge_stage(packed2d, gtab.reshape(NBOND * NN, H),
                                   hin, batch, True)
        hin, gtab, vn = _tcmid_stage(
            hin, agg, pool, oh, c,
            p['adapt_W'][l], r2(p['adapt_b'][l]), eps[l],
            p['conv_W1'][l], r2(p['conv_b1'][l]),
            r2(p['conv_ln_g'][l]), r2(p['conv_ln_b'][l]),
            p['conv_W2'][l], r2(p['conv_b2'][l]),
            p['vn_W1'][l], r2(p['vn_b1'][l]),
            r2(p['vn_ln_g'][l]), r2(p['vn_ln_b'][l]),
            p['vn_W2'][l], r2(p['vn_b2'][l]),
            vn, bond[l + 1])

    agg, _ = _sc_edge_stage(packed2d, gtab.reshape(NBOND * NN, H),
                            hin, batch, False)
    o = _tcfin_stage(
        hin, agg, oh, c,
        p['adapt_W'][L - 1], r2(p['adapt_b'][L - 1]), eps[L - 1],
        p['conv_W1'][L - 1], r2(p['conv_b1'][L - 1]),
        r2(p['conv_ln_g'][L - 1]), r2(p['conv_ln_b'][L - 1]),
        p['conv_W2'][L - 1], r2(p['conv_b2'][L - 1]),
        p['dec_W1'], r2(p['dec_b1']),
        r2(p['dec_ln_g']), r2(p['dec_ln_b']),
        p['dec_W2'], r2(p['dec_b2']))
    return o
